# baseline (speedup 1.0000x reference)
import functools

import jax
import jax.numpy as jnp
import numpy as np
from jax import lax
from jax.experimental import pallas as pl
from jax.experimental.pallas import tpu as pltpu

_NT = (((1,), (1,)), ((), ()))


def _cast_body(w_ref, o_ref, *, q_chunks, scale):
    if q_chunks:
        s = jnp.where(pl.program_id(0) < q_chunks, scale, 1.0).astype(jnp.float32)
        o_ref[...] = (w_ref[...] * s).astype(jnp.bfloat16)
    else:
        o_ref[...] = w_ref[...].astype(jnp.bfloat16)


def _cast_bf16(w, rows_per_chunk, q_chunks=0, scale=1.0):
    R, C = w.shape
    grid = R // rows_per_chunk
    return pl.pallas_call(
        functools.partial(_cast_body, q_chunks=q_chunks, scale=scale),
        out_shape=jax.ShapeDtypeStruct((R, C), jnp.bfloat16),
        grid=(grid,),
        in_specs=[pl.BlockSpec((rows_per_chunk, C), lambda i: (i, 0))],
        out_specs=pl.BlockSpec((rows_per_chunk, C), lambda i: (i, 0)),
        compiler_params=pltpu.CompilerParams(
            dimension_semantics=("parallel",),
        ),
    )(w)


def _layer_kernel(x_hbm, pos_hbm, wqkv_ref, bqkv_ref, wo_ref, bo_ref,
                  w1_ref, b1_ref, w2_ref, b2_ref, out_hbm,
                  x_vm, pos_vm, out_vm, ctx_ref, sx, sp, so,
                  *, nhead, head_dim, scale, nsteps, seq):
    f32 = jnp.float32
    bf16 = jnp.bfloat16
    D = nhead * head_dim
    g = pl.program_id(0)
    slot = lax.rem(g, 3)
    oslot = lax.rem(g, 2)
    onslot = lax.rem(g + 1, 2)
    S2 = 2 * seq

    def in_copies(gi, si):
        b0 = 2 * gi
        return (
            pltpu.make_async_copy(x_hbm.at[:, b0, :], x_vm.at[si, 0:seq, :], sx.at[si, 0]),
            pltpu.make_async_copy(x_hbm.at[:, b0 + 1, :], x_vm.at[si, seq:S2, :], sx.at[si, 1]),
            pltpu.make_async_copy(pos_hbm.at[:, b0, :], pos_vm.at[si, 0:seq, :], sp.at[si, 0]),
            pltpu.make_async_copy(pos_hbm.at[:, b0 + 1, :], pos_vm.at[si, seq:S2, :], sp.at[si, 1]),
        )

    def out_copies(gi, si):
        b0 = 2 * gi
        return (
            pltpu.make_async_copy(out_vm.at[si, 0:seq, :], out_hbm.at[:, b0, :], so.at[si, 0]),
            pltpu.make_async_copy(out_vm.at[si, seq:S2, :], out_hbm.at[:, b0 + 1, :], so.at[si, 1]),
        )

    @pl.when(g == 0)
    def _():
        for c in in_copies(0, 0):
            c.start()
        if nsteps >= 2:
            for c in in_copies(1, 1):
                c.start()

    @pl.when(g + 2 < nsteps)
    def _():
        for c in in_copies(g + 2, lax.rem(g + 2, 3)):
            c.start()

    for c in in_copies(g, slot):
        c.wait()

    x = x_vm[slot] + pos_vm[slot]

    lane = lax.broadcasted_iota(jnp.int32, (1, 3 * D), 1)
    bqkv = jnp.where(lane < D, bqkv_ref[...] * scale, bqkv_ref[...])

    qkv = lax.dot_general(x.astype(bf16), wqkv_ref[...], _NT,
                          preferred_element_type=f32) + bqkv
    qkv_bf = qkv.astype(bf16)

    for half in range(2):
        r0, r1 = half * seq, (half + 1) * seq
        for h in range(nhead):
            q = qkv_bf[r0:r1, h * head_dim:(h + 1) * head_dim]
            k = qkv_bf[r0:r1, D + h * head_dim:D + (h + 1) * head_dim]
            v = qkv_bf[r0:r1, 2 * D + h * head_dim:2 * D + (h + 1) * head_dim]

            s = lax.dot_general(q, k, _NT, preferred_element_type=f32)
            p = jnp.exp(s)
            denom = jnp.sum(p, axis=-1, keepdims=True)
            ctx = jnp.dot(p.astype(bf16), v, preferred_element_type=f32)
            ctx = ctx * pl.reciprocal(denom, approx=True)
            ctx_ref[r0:r1, h * head_dim:(h + 1) * head_dim] = ctx.astype(bf16)

    attn = lax.dot_general(ctx_ref[...], wo_ref[...], _NT,
                           preferred_element_type=f32) + bo_ref[...]
    x1 = x + attn

    h1 = lax.dot_general(x1.astype(bf16), w1_ref[...], _NT,
                         preferred_element_type=f32) + b1_ref[...]
    h1 = jnp.maximum(h1, 0.0)
    ff = lax.dot_general(h1.astype(bf16), w2_ref[...], _NT,
                         preferred_element_type=f32) + b2_ref[...]

    @pl.when(g >= 2)
    def _():
        for c in out_copies(g - 2, oslot):
            c.wait()

    out_vm[oslot] = (x1 + ff).astype(out_vm.dtype)
    for c in out_copies(g, oslot):
        c.start()

    if nsteps >= 2:
        @pl.when(g == nsteps - 1)
        def _():
            for c in out_copies(g - 1, onslot):
                c.wait()
            for c in out_copies(g, oslot):
                c.wait()
    else:
        for c in out_copies(g, oslot):
            c.wait()


def kernel(queries, pos_emb, wqkv, bqkv, wo, bo, w1, b1, w2, b2):
    S, B, D = queries.shape
    nhead = 16
    hd = D // nhead
    FF = w1.shape[0]
    scale = 1.0 / float(np.sqrt(hd))
    assert B % 2 == 0
    nsteps = B // 2

    qc = min(256, D)
    wqkv_bf = _cast_bf16(wqkv, qc, q_chunks=D // qc, scale=scale)
    wo_bf = _cast_bf16(wo, min(256, D))
    w1_bf = _cast_bf16(w1, min(512, FF))
    w2_bf = _cast_bf16(w2, min(128, D))

    body = functools.partial(_layer_kernel, nhead=nhead, head_dim=hd,
                             scale=scale, nsteps=nsteps, seq=S)

    def _call(single_buffer):
        def const_spec(shape):
            if single_buffer:
                return pl.BlockSpec(shape, lambda g: (0, 0), pipeline_mode=pl.Buffered(1))
            return pl.BlockSpec(shape, lambda g: (0, 0))

        any_spec = pl.BlockSpec(memory_space=pl.ANY)
        return pl.pallas_call(
            body,
            out_shape=jax.ShapeDtypeStruct((S, B, D), queries.dtype),
            grid_spec=pltpu.PrefetchScalarGridSpec(
                num_scalar_prefetch=0,
                grid=(nsteps,),
                in_specs=[
                    any_spec,
                    any_spec,
                    const_spec((3 * D, D)),
                    const_spec((1, 3 * D)),
                    const_spec((D, D)),
                    const_spec((1, D)),
                    const_spec((FF, D)),
                    const_spec((1, FF)),
                    const_spec((D, FF)),
                    const_spec((1, D)),
                ],
                out_specs=pl.BlockSpec(memory_space=pl.ANY),
                scratch_shapes=[
                    pltpu.VMEM((3, 2 * S, D), jnp.float32),
                    pltpu.VMEM((3, 2 * S, D), jnp.float32),
                    pltpu.VMEM((2, 2 * S, D), jnp.float32),
                    pltpu.VMEM((2 * S, D), jnp.bfloat16),
                    pltpu.SemaphoreType.DMA((3, 2)),
                    pltpu.SemaphoreType.DMA((3, 2)),
                    pltpu.SemaphoreType.DMA((2, 2)),
                ],
            ),
            compiler_params=pltpu.CompilerParams(
                dimension_semantics=("arbitrary",),
                vmem_limit_bytes=60000 * 1024,
            ),
        )(queries, pos_emb, wqkv_bf, bqkv.astype(jnp.float32), wo_bf,
          bo.astype(jnp.float32), w1_bf, b1.astype(jnp.float32), w2_bf,
          b2.astype(jnp.float32))

    try:
        return _call(True)
    except Exception:
        return _call(False)


# speedup vs baseline: 1.0160x; 1.0160x over previous
import functools

import jax
import jax.numpy as jnp
import numpy as np
from jax import lax
from jax.experimental import pallas as pl
from jax.experimental.pallas import tpu as pltpu

_NT = (((1,), (1,)), ((), ()))


def _cast_all_body(wqkv_ref, wo_ref, w1_ref, w2_ref,
                   oqkv_ref, oo_ref, o1_ref, o2_ref, *, cq, d_model, scale):
    rows = cq * pl.program_id(0) + lax.broadcasted_iota(
        jnp.int32, (wqkv_ref.shape[0], 1), 0)
    s = jnp.where(rows < d_model, jnp.float32(scale), jnp.float32(1.0))
    oqkv_ref[...] = (wqkv_ref[...] * s).astype(jnp.bfloat16)
    oo_ref[...] = wo_ref[...].astype(jnp.bfloat16)
    o1_ref[...] = w1_ref[...].astype(jnp.bfloat16)
    o2_ref[...] = w2_ref[...].astype(jnp.bfloat16)


def _cast_weights(wqkv, wo, w1, w2, scale):
    (Rq, C), (Ro, _), (R1, _), (R2, C2) = wqkv.shape, wo.shape, w1.shape, w2.shape
    n = 12 if Rq % 12 == 0 and Ro % 4 == 0 and R1 % 8 == 0 and R2 % 8 == 0 else 1
    cq, co, c1, c2 = Rq // n, Ro // max(n // 3, 1), R1 // max(2 * n // 3, 1), R2 // max(2 * n // 3, 1)
    nq, no, n1, n2 = Rq // cq, Ro // co, R1 // c1, R2 // c2

    def clamp(m):
        return lambda i: (jnp.minimum(i, m - 1), 0)

    return pl.pallas_call(
        functools.partial(_cast_all_body, cq=cq, d_model=Rq // 3, scale=scale),
        out_shape=(jax.ShapeDtypeStruct((Rq, C), jnp.bfloat16),
                   jax.ShapeDtypeStruct((Ro, C), jnp.bfloat16),
                   jax.ShapeDtypeStruct((R1, C), jnp.bfloat16),
                   jax.ShapeDtypeStruct((R2, C2), jnp.bfloat16)),
        grid=(n,),
        in_specs=[pl.BlockSpec((cq, C), clamp(nq)),
                  pl.BlockSpec((co, C), clamp(no)),
                  pl.BlockSpec((c1, C), clamp(n1)),
                  pl.BlockSpec((c2, C2), clamp(n2))],
        out_specs=(pl.BlockSpec((cq, C), clamp(nq)),
                   pl.BlockSpec((co, C), clamp(no)),
                   pl.BlockSpec((c1, C), clamp(n1)),
                   pl.BlockSpec((c2, C2), clamp(n2))),
        compiler_params=pltpu.CompilerParams(
            dimension_semantics=("arbitrary",),
        ),
    )(wqkv, wo, w1, w2)


def _layer_kernel(x_hbm, pos_hbm, wqkv_ref, bqkv_ref, wo_ref, bo_ref,
                  w1_ref, b1_ref, w2_ref, b2_ref, out_hbm,
                  x_vm, pos_vm, out_vm, ctx_ref, sx, sp, so,
                  *, nhead, head_dim, scale, nsteps, seq):
    f32 = jnp.float32
    bf16 = jnp.bfloat16
    D = nhead * head_dim
    g = pl.program_id(0)
    slot = lax.rem(g, 3)
    oslot = lax.rem(g, 2)
    onslot = lax.rem(g + 1, 2)
    S2 = 2 * seq

    def in_copies(gi, si):
        b0 = 2 * gi
        return (
            pltpu.make_async_copy(x_hbm.at[:, b0, :], x_vm.at[si, 0:seq, :], sx.at[si, 0]),
            pltpu.make_async_copy(x_hbm.at[:, b0 + 1, :], x_vm.at[si, seq:S2, :], sx.at[si, 1]),
            pltpu.make_async_copy(pos_hbm.at[:, b0, :], pos_vm.at[si, 0:seq, :], sp.at[si, 0]),
            pltpu.make_async_copy(pos_hbm.at[:, b0 + 1, :], pos_vm.at[si, seq:S2, :], sp.at[si, 1]),
        )

    def out_copies(gi, si):
        b0 = 2 * gi
        return (
            pltpu.make_async_copy(out_vm.at[si, 0:seq, :], out_hbm.at[:, b0, :], so.at[si, 0]),
            pltpu.make_async_copy(out_vm.at[si, seq:S2, :], out_hbm.at[:, b0 + 1, :], so.at[si, 1]),
        )

    @pl.when(g == 0)
    def _():
        for c in in_copies(0, 0):
            c.start()
        if nsteps >= 2:
            for c in in_copies(1, 1):
                c.start()

    @pl.when(g + 2 < nsteps)
    def _():
        for c in in_copies(g + 2, lax.rem(g + 2, 3)):
            c.start()

    for c in in_copies(g, slot):
        c.wait()

    x = x_vm[slot] + pos_vm[slot]

    lane = lax.broadcasted_iota(jnp.int32, (1, 3 * D), 1)
    bqkv = jnp.where(lane < D, bqkv_ref[...] * scale, bqkv_ref[...])

    qkv = lax.dot_general(x.astype(bf16), wqkv_ref[...], _NT,
                          preferred_element_type=f32) + bqkv
    qkv_bf = qkv.astype(bf16)

    for half in range(2):
        r0, r1 = half * seq, (half + 1) * seq
        for h in range(nhead):
            q = qkv_bf[r0:r1, h * head_dim:(h + 1) * head_dim]
            k = qkv_bf[r0:r1, D + h * head_dim:D + (h + 1) * head_dim]
            v = qkv_bf[r0:r1, 2 * D + h * head_dim:2 * D + (h + 1) * head_dim]

            s = lax.dot_general(q, k, _NT, preferred_element_type=f32)
            p = jnp.exp(s)
            denom = jnp.sum(p, axis=-1, keepdims=True)
            ctx = jnp.dot(p.astype(bf16), v, preferred_element_type=f32)
            ctx = ctx * pl.reciprocal(denom, approx=True)
            ctx_ref[r0:r1, h * head_dim:(h + 1) * head_dim] = ctx.astype(bf16)

    attn = lax.dot_general(ctx_ref[...], wo_ref[...], _NT,
                           preferred_element_type=f32) + bo_ref[...]
    x1 = x + attn

    h1 = lax.dot_general(x1.astype(bf16), w1_ref[...], _NT,
                         preferred_element_type=f32) + b1_ref[...]
    h1 = jnp.maximum(h1, 0.0)
    ff = lax.dot_general(h1.astype(bf16), w2_ref[...], _NT,
                         preferred_element_type=f32) + b2_ref[...]

    @pl.when(g >= 2)
    def _():
        for c in out_copies(g - 2, oslot):
            c.wait()

    out_vm[oslot] = (x1 + ff).astype(out_vm.dtype)
    for c in out_copies(g, oslot):
        c.start()

    if nsteps >= 2:
        @pl.when(g == nsteps - 1)
        def _():
            for c in out_copies(g - 1, onslot):
                c.wait()
            for c in out_copies(g, oslot):
                c.wait()
    else:
        for c in out_copies(g, oslot):
            c.wait()


def kernel(queries, pos_emb, wqkv, bqkv, wo, bo, w1, b1, w2, b2):
    S, B, D = queries.shape
    nhead = 16
    hd = D // nhead
    FF = w1.shape[0]
    scale = 1.0 / float(np.sqrt(hd))
    assert B % 2 == 0
    nsteps = B // 2

    wqkv_bf, wo_bf, w1_bf, w2_bf = _cast_weights(wqkv, wo, w1, w2, scale)

    body = functools.partial(_layer_kernel, nhead=nhead, head_dim=hd,
                             scale=scale, nsteps=nsteps, seq=S)

    def _call(single_buffer):
        def const_spec(shape):
            if single_buffer:
                return pl.BlockSpec(shape, lambda g: (0, 0), pipeline_mode=pl.Buffered(1))
            return pl.BlockSpec(shape, lambda g: (0, 0))

        any_spec = pl.BlockSpec(memory_space=pl.ANY)
        return pl.pallas_call(
            body,
            out_shape=jax.ShapeDtypeStruct((S, B, D), queries.dtype),
            grid_spec=pltpu.PrefetchScalarGridSpec(
                num_scalar_prefetch=0,
                grid=(nsteps,),
                in_specs=[
                    any_spec,
                    any_spec,
                    const_spec((3 * D, D)),
                    const_spec((1, 3 * D)),
                    const_spec((D, D)),
                    const_spec((1, D)),
                    const_spec((FF, D)),
                    const_spec((1, FF)),
                    const_spec((D, FF)),
                    const_spec((1, D)),
                ],
                out_specs=pl.BlockSpec(memory_space=pl.ANY),
                scratch_shapes=[
                    pltpu.VMEM((3, 2 * S, D), jnp.float32),
                    pltpu.VMEM((3, 2 * S, D), jnp.float32),
                    pltpu.VMEM((2, 2 * S, D), jnp.float32),
                    pltpu.VMEM((2 * S, D), jnp.bfloat16),
                    pltpu.SemaphoreType.DMA((3, 2)),
                    pltpu.SemaphoreType.DMA((3, 2)),
                    pltpu.SemaphoreType.DMA((2, 2)),
                ],
            ),
            compiler_params=pltpu.CompilerParams(
                dimension_semantics=("arbitrary",),
                vmem_limit_bytes=60000 * 1024,
            ),
        )(queries, pos_emb, wqkv_bf, bqkv.astype(jnp.float32), wo_bf,
          bo.astype(jnp.float32), w1_bf, b1.astype(jnp.float32), w2_bf,
          b2.astype(jnp.float32))

    try:
        return _call(True)
    except Exception:
        return _call(False)
